# Initial kernel scaffold; baseline (speedup 1.0000x reference)
import jax
import jax.numpy as jnp
from jax.experimental import pallas as pl


def kernel(x, edge_index, W1, b1, W2, b2):
    raise NotImplementedError("write your pallas kernel here")


# speedup vs baseline: 8.3012x; 8.3012x over previous
import functools
import math

import jax
import jax.numpy as jnp
from jax import lax
from jax.experimental import pallas as pl
from jax.experimental.pallas import tpu as pltpu
from jax.experimental.pallas import tpu_sc as plsc

CH = 128
NC = 2
NS = 16
NW = NC * NS


def _stripe_chunks(stripe):
    out = []
    off = 0
    while off < stripe:
        ln = min(CH, stripe - off)
        out.append((off, ln))
        off += ln
    return out


def _make_sc_agg(n_acc, k_per_tile, width, gather):
    stripe = n_acc // NS
    chunks = _stripe_chunks(stripe)
    mesh = plsc.VectorSubcoreMesh(core_axis_name="c", subcore_axis_name="s")

    scratch = [
        pltpu.VMEM((k_per_tile, CH), jnp.int32),
        pltpu.VMEM((CH, width), jnp.float32),
        pltpu.VMEM_SHARED((n_acc, width), jnp.float32),
        pltpu.SemaphoreType.DMA,
    ]
    if gather:
        scratch.insert(1, pltpu.VMEM((k_per_tile, CH), jnp.int32))

    def body_common(src_hbm, dst_hbm, table_hbm, zeros_hbm, out_hbm,
                    idxd_v, idxs_v, rows_v, acc, sem):
        c = lax.axis_index("c")
        s = lax.axis_index("s")
        wid = c * NS + s
        base = wid * k_per_tile
        stripe0 = s * stripe

        pltpu.sync_copy(zeros_hbm, rows_v)
        for off, ln in chunks:
            pltpu.sync_copy(rows_v.at[pl.ds(0, ln)],
                            acc.at[pl.ds(stripe0 + off, ln)])

        pltpu.sync_copy(dst_hbm.at[pl.ds(base, k_per_tile)], idxd_v)
        if gather:
            pltpu.sync_copy(src_hbm.at[pl.ds(base, k_per_tile)], idxs_v)
        else:
            pltpu.sync_copy(table_hbm, rows_v)

        plsc.subcore_barrier()

        def step(j, carry):
            if gather:
                pltpu.async_copy(table_hbm.at[idxs_v.at[j]], rows_v, sem).wait()
            pltpu.sync_copy(rows_v, acc.at[idxd_v.at[j]], add=True)
            return carry

        lax.fori_loop(0, k_per_tile, step, 0)

        plsc.subcore_barrier()

        out_base = c * n_acc + stripe0
        for off, ln in chunks:
            pltpu.sync_copy(acc.at[pl.ds(stripe0 + off, ln)],
                            rows_v.at[pl.ds(0, ln)])
            pltpu.sync_copy(rows_v.at[pl.ds(0, ln)],
                            out_hbm.at[pl.ds(out_base + off, ln)])

    if gather:
        def body(src_hbm, dst_hbm, table_hbm, zeros_hbm, out_hbm,
                 idxd_v, idxs_v, rows_v, acc, sem):
            body_common(src_hbm, dst_hbm, table_hbm, zeros_hbm, out_hbm,
                        idxd_v, idxs_v, rows_v, acc, sem)
    else:
        def body(dst_hbm, table_hbm, zeros_hbm, out_hbm,
                 idxd_v, rows_v, acc, sem):
            body_common(None, dst_hbm, table_hbm, zeros_hbm, out_hbm,
                        idxd_v, None, rows_v, acc, sem)

    return pl.kernel(
        body,
        out_type=jax.ShapeDtypeStruct((NC * n_acc, width), jnp.float32),
        mesh=mesh,
        scratch_types=scratch,
    )


def _dinv_from(degs):
    deg = degs[0, :, 0:1] + degs[1, :, 0:1] + 1.0
    return lax.rsqrt(deg)


def _tc_b_body(x_ref, w1_ref, degs_ref, g1_ref):
    dinv = _dinv_from(degs_ref[...])
    h = jnp.dot(x_ref[...], w1_ref[...],
                preferred_element_type=jnp.float32,
                precision=lax.Precision.HIGHEST)
    g1_ref[...] = dinv * h


def _tc_d_body(p_ref, g1_ref, degs_ref, w2_ref, b1_ref, g2_ref):
    dinv = _dinv_from(degs_ref[...])
    p = p_ref[...]
    agg = p[0] + p[1] + g1_ref[...]
    out1 = dinv * agg + b1_ref[...]
    h2 = jnp.dot(out1, w2_ref[...],
                 preferred_element_type=jnp.float32,
                 precision=lax.Precision.HIGHEST)
    g2_ref[...] = dinv * h2


def _tc_f_body(q_ref, g2_ref, degs_ref, b2_ref, out_ref):
    dinv = _dinv_from(degs_ref[...])
    q = q_ref[...]
    d_out = out_ref.shape[1]
    z = (dinv * (q[0] + q[1] + g2_ref[...]) + b2_ref[...])[:, :d_out]
    zmax = jnp.max(z, axis=1, keepdims=True)
    zmin = jnp.min(z, axis=1, keepdims=True)
    s = (z - zmin) / (zmax - zmin)
    nrm = jnp.sqrt(jnp.sum(s * s, axis=1, keepdims=True))
    out_ref[...] = s / jnp.maximum(nrm, 1e-12)


@jax.jit
def kernel(x, edge_index, W1, b1, W2, b2):
    n, d_in = x.shape
    d_hid = W1.shape[1]
    d_out = W2.shape[1]
    e = edge_index.shape[1]

    k_per_tile = -(-(-(-e // (CH * NW))) // 8) * 8
    e_pad = k_per_tile * CH * NW
    stripe = -(-(n + 112) // (NS * 8)) * 8
    n_acc = NS * stripe
    pad_rows = n_acc - n

    src = edge_index[0].astype(jnp.int32)
    dst = edge_index[1].astype(jnp.int32)
    npad = e_pad - e
    src_p = jnp.concatenate([src, jnp.zeros((npad,), jnp.int32)])
    dst_p = jnp.concatenate(
        [dst, n + (jnp.arange(npad, dtype=jnp.int32) % pad_rows)])
    src2 = src_p.reshape(NW * k_per_tile, CH)
    dst2 = dst_p.reshape(NW * k_per_tile, CH)

    dp = 128
    W2p = jnp.zeros((d_hid, dp), jnp.float32).at[:, :d_out].set(W2)
    b2p = jnp.zeros((1, dp), jnp.float32).at[0, :d_out].set(b2)

    ones_blk = jnp.zeros((CH, dp), jnp.float32).at[:, 0].set(1.0)
    zeros_h = jnp.zeros((CH, d_hid), jnp.float32)
    zeros_p = jnp.zeros((CH, dp), jnp.float32)

    deg_k = _make_sc_agg(n_acc, k_per_tile, dp, gather=False)
    degs = deg_k(dst2, ones_blk, zeros_p).reshape(NC, n_acc, dp)

    bk = 1000
    grid = (n // bk,)
    g1 = pl.pallas_call(
        _tc_b_body,
        grid=grid,
        in_specs=[
            pl.BlockSpec((bk, d_in), lambda i: (i, 0)),
            pl.BlockSpec((d_in, d_hid), lambda i: (0, 0)),
            pl.BlockSpec((NC, bk, 128), lambda i: (0, i, 0)),
        ],
        out_specs=pl.BlockSpec((bk, d_hid), lambda i: (i, 0)),
        out_shape=jax.ShapeDtypeStruct((n, d_hid), jnp.float32),
    )(x, W1, degs)

    agg1_k = _make_sc_agg(n_acc, k_per_tile, d_hid, gather=True)
    P = agg1_k(src2, dst2, g1, zeros_h).reshape(NC, n_acc, d_hid)

    g2 = pl.pallas_call(
        _tc_d_body,
        grid=grid,
        in_specs=[
            pl.BlockSpec((NC, bk, d_hid), lambda i: (0, i, 0)),
            pl.BlockSpec((bk, d_hid), lambda i: (i, 0)),
            pl.BlockSpec((NC, bk, 128), lambda i: (0, i, 0)),
            pl.BlockSpec((d_hid, dp), lambda i: (0, 0)),
            pl.BlockSpec((1, d_hid), lambda i: (0, 0)),
        ],
        out_specs=pl.BlockSpec((bk, dp), lambda i: (i, 0)),
        out_shape=jax.ShapeDtypeStruct((n, dp), jnp.float32),
    )(P, g1, degs, W2p, b1.reshape(1, d_hid))

    agg2_k = _make_sc_agg(n_acc, k_per_tile, dp, gather=True)
    Q = agg2_k(src2, dst2, g2, zeros_p).reshape(NC, n_acc, dp)

    out = pl.pallas_call(
        _tc_f_body,
        grid=grid,
        in_specs=[
            pl.BlockSpec((NC, bk, dp), lambda i: (0, i, 0)),
            pl.BlockSpec((bk, dp), lambda i: (i, 0)),
            pl.BlockSpec((NC, bk, 128), lambda i: (0, i, 0)),
            pl.BlockSpec((1, dp), lambda i: (0, 0)),
        ],
        out_specs=pl.BlockSpec((bk, d_out), lambda i: (i, 0)),
        out_shape=jax.ShapeDtypeStruct((n, d_out), jnp.float32),
    )(Q, g2, degs, b2p)

    return out


# speedup vs baseline: 8.9159x; 1.0741x over previous
import functools
import math

import jax
import jax.numpy as jnp
from jax import lax
from jax.experimental import pallas as pl
from jax.experimental.pallas import tpu as pltpu
from jax.experimental.pallas import tpu_sc as plsc

CH = 128
NC = 2
NS = 16
NW = NC * NS


def _stripe_chunks(stripe):
    out = []
    off = 0
    while off < stripe:
        ln = min(CH, stripe - off)
        out.append((off, ln))
        off += ln
    return out


RING = 2


def _make_sc_agg(n_acc, k_per_tile, width, gather):
    stripe = n_acc // NS
    chunks = _stripe_chunks(stripe)
    mesh = plsc.VectorSubcoreMesh(core_axis_name="c", subcore_axis_name="s")
    nbuf = RING if gather else 1
    idxb = k_per_tile // 2 if gather else k_per_tile
    assert idxb % RING == 0 and k_per_tile % 8 == 0

    scratch = [pltpu.VMEM((idxb, CH), jnp.int32)]
    if gather:
        scratch.append(pltpu.VMEM((idxb, CH), jnp.int32))
    scratch += [pltpu.VMEM((CH, width), jnp.float32)] * nbuf
    scratch.append(pltpu.VMEM_SHARED((n_acc, width), jnp.float32))
    scratch += [pltpu.SemaphoreType.DMA] * (2 * nbuf)

    def body_common(src_hbm, dst_hbm, table_hbm, zeros_hbm, out_hbm,
                    idxd_v, idxs_v, bufs, acc, gsem, ssem):
        c = lax.axis_index("c")
        s = lax.axis_index("s")
        wid = c * NS + s
        base = wid * k_per_tile
        stripe0 = s * stripe

        pltpu.sync_copy(zeros_hbm, bufs[0])
        for off, ln in chunks:
            pltpu.sync_copy(bufs[0].at[pl.ds(0, ln)],
                            acc.at[pl.ds(stripe0 + off, ln)])

        def drain(buf, sem):
            pltpu.make_async_copy(zeros_hbm, buf, sem).wait()

        if gather:
            plsc.subcore_barrier()
            for blk in range(k_per_tile // idxb):
                blk_base = base + blk * idxb
                pltpu.sync_copy(dst_hbm.at[pl.ds(blk_base, idxb)], idxd_v)
                pltpu.sync_copy(src_hbm.at[pl.ds(blk_base, idxb)], idxs_v)

                for b in range(RING):
                    pltpu.async_copy(table_hbm.at[idxs_v.at[b]],
                                     bufs[b], gsem[b])

                def step(g, carry):
                    j0 = g * RING
                    for b in range(RING):
                        drain(bufs[b], gsem[b])
                        pltpu.async_copy(bufs[b], acc.at[idxd_v.at[j0 + b]],
                                         ssem[b], add=True)
                    for b in range(RING):
                        drain(bufs[b], ssem[b])
                        pltpu.async_copy(table_hbm.at[idxs_v.at[j0 + RING + b]],
                                         bufs[b], gsem[b])
                    return carry

                lax.fori_loop(0, idxb // RING - 1, step, 0)
                j0 = idxb - RING
                for b in range(RING):
                    drain(bufs[b], gsem[b])
                    pltpu.async_copy(bufs[b], acc.at[idxd_v.at[j0 + b]],
                                     ssem[b], add=True)
                for b in range(RING):
                    drain(bufs[b], ssem[b])
        else:
            pltpu.sync_copy(dst_hbm.at[pl.ds(base, k_per_tile)], idxd_v)
            pltpu.sync_copy(table_hbm, bufs[0])
            plsc.subcore_barrier()
            fire = 8

            def step(g, carry):
                j0 = g * fire
                for b in range(fire):
                    pltpu.async_copy(bufs[0], acc.at[idxd_v.at[j0 + b]],
                                     ssem[0], add=True)
                for b in range(fire):
                    drain(bufs[0], ssem[0])
                return carry

            lax.fori_loop(0, k_per_tile // fire, step, 0)

        plsc.subcore_barrier()

        out_base = c * n_acc + stripe0
        for off, ln in chunks:
            pltpu.sync_copy(acc.at[pl.ds(stripe0 + off, ln)],
                            bufs[0].at[pl.ds(0, ln)])
            pltpu.sync_copy(bufs[0].at[pl.ds(0, ln)],
                            out_hbm.at[pl.ds(out_base + off, ln)])

    if gather:
        def body(src_hbm, dst_hbm, table_hbm, zeros_hbm, out_hbm,
                 idxd_v, idxs_v, b0, b1, acc, g0, g1, s0, s1):
            body_common(src_hbm, dst_hbm, table_hbm, zeros_hbm, out_hbm,
                        idxd_v, idxs_v, [b0, b1], acc, [g0, g1], [s0, s1])
    else:
        def body(dst_hbm, table_hbm, zeros_hbm, out_hbm,
                 idxd_v, b0, acc, g0, s0):
            body_common(None, dst_hbm, table_hbm, zeros_hbm, out_hbm,
                        idxd_v, None, [b0], acc, [g0], [s0])

    return pl.kernel(
        body,
        out_type=jax.ShapeDtypeStruct((NC * n_acc, width), jnp.float32),
        mesh=mesh,
        scratch_types=scratch,
    )


def _dinv_from(degs):
    deg = degs[0, :, 0:1] + degs[1, :, 0:1] + 1.0
    return lax.rsqrt(deg)


def _tc_b_body(x_ref, w1_ref, degs_ref, g1_ref):
    dinv = _dinv_from(degs_ref[...])
    h = jnp.dot(x_ref[...], w1_ref[...],
                preferred_element_type=jnp.float32,
                precision=lax.Precision.HIGHEST)
    g1_ref[...] = dinv * h


def _tc_d_body(p_ref, g1_ref, degs_ref, w2_ref, b1_ref, g2_ref):
    dinv = _dinv_from(degs_ref[...])
    p = p_ref[...]
    agg = p[0] + p[1] + g1_ref[...]
    out1 = dinv * agg + b1_ref[...]
    h2 = jnp.dot(out1, w2_ref[...],
                 preferred_element_type=jnp.float32,
                 precision=lax.Precision.HIGHEST)
    g2_ref[...] = dinv * h2


def _tc_f_body(q_ref, g2_ref, degs_ref, b2_ref, out_ref):
    dinv = _dinv_from(degs_ref[...])
    q = q_ref[...]
    d_out = out_ref.shape[1]
    z = (dinv * (q[0] + q[1] + g2_ref[...]) + b2_ref[...])[:, :d_out]
    zmax = jnp.max(z, axis=1, keepdims=True)
    zmin = jnp.min(z, axis=1, keepdims=True)
    s = (z - zmin) / (zmax - zmin)
    nrm = jnp.sqrt(jnp.sum(s * s, axis=1, keepdims=True))
    out_ref[...] = s / jnp.maximum(nrm, 1e-12)


@jax.jit
def kernel(x, edge_index, W1, b1, W2, b2):
    n, d_in = x.shape
    d_hid = W1.shape[1]
    d_out = W2.shape[1]
    e = edge_index.shape[1]

    k_per_tile = -(-(-(-e // (CH * NW))) // 8) * 8
    e_pad = k_per_tile * CH * NW
    stripe = -(-(n + 112) // (NS * 8)) * 8
    n_acc = NS * stripe
    pad_rows = n_acc - n

    src = edge_index[0].astype(jnp.int32)
    dst = edge_index[1].astype(jnp.int32)
    npad = e_pad - e
    src_p = jnp.concatenate([src, jnp.zeros((npad,), jnp.int32)])
    dst_p = jnp.concatenate(
        [dst, n + (jnp.arange(npad, dtype=jnp.int32) % pad_rows)])
    src2 = src_p.reshape(NW * k_per_tile, CH)
    dst2 = dst_p.reshape(NW * k_per_tile, CH)

    dp = 128
    W2p = jnp.zeros((d_hid, dp), jnp.float32).at[:, :d_out].set(W2)
    b2p = jnp.zeros((1, dp), jnp.float32).at[0, :d_out].set(b2)

    ones_blk = jnp.zeros((CH, dp), jnp.float32).at[:, 0].set(1.0)
    zeros_h = jnp.zeros((CH, d_hid), jnp.float32)
    zeros_p = jnp.zeros((CH, dp), jnp.float32)

    deg_k = _make_sc_agg(n_acc, k_per_tile, dp, gather=False)
    degs = deg_k(dst2, ones_blk, zeros_p).reshape(NC, n_acc, dp)

    bk = 1000
    grid = (n // bk,)
    g1 = pl.pallas_call(
        _tc_b_body,
        grid=grid,
        in_specs=[
            pl.BlockSpec((bk, d_in), lambda i: (i, 0)),
            pl.BlockSpec((d_in, d_hid), lambda i: (0, 0)),
            pl.BlockSpec((NC, bk, 128), lambda i: (0, i, 0)),
        ],
        out_specs=pl.BlockSpec((bk, d_hid), lambda i: (i, 0)),
        out_shape=jax.ShapeDtypeStruct((n, d_hid), jnp.float32),
    )(x, W1, degs)

    agg1_k = _make_sc_agg(n_acc, k_per_tile, d_hid, gather=True)
    P = agg1_k(src2, dst2, g1, zeros_h).reshape(NC, n_acc, d_hid)

    g2 = pl.pallas_call(
        _tc_d_body,
        grid=grid,
        in_specs=[
            pl.BlockSpec((NC, bk, d_hid), lambda i: (0, i, 0)),
            pl.BlockSpec((bk, d_hid), lambda i: (i, 0)),
            pl.BlockSpec((NC, bk, 128), lambda i: (0, i, 0)),
            pl.BlockSpec((d_hid, dp), lambda i: (0, 0)),
            pl.BlockSpec((1, d_hid), lambda i: (0, 0)),
        ],
        out_specs=pl.BlockSpec((bk, dp), lambda i: (i, 0)),
        out_shape=jax.ShapeDtypeStruct((n, dp), jnp.float32),
    )(P, g1, degs, W2p, b1.reshape(1, d_hid))

    agg2_k = _make_sc_agg(n_acc, k_per_tile, dp, gather=True)
    Q = agg2_k(src2, dst2, g2, zeros_p).reshape(NC, n_acc, dp)

    out = pl.pallas_call(
        _tc_f_body,
        grid=grid,
        in_specs=[
            pl.BlockSpec((NC, bk, dp), lambda i: (0, i, 0)),
            pl.BlockSpec((bk, dp), lambda i: (i, 0)),
            pl.BlockSpec((NC, bk, 128), lambda i: (0, i, 0)),
            pl.BlockSpec((1, dp), lambda i: (0, 0)),
        ],
        out_specs=pl.BlockSpec((bk, d_out), lambda i: (i, 0)),
        out_shape=jax.ShapeDtypeStruct((n, d_out), jnp.float32),
    )(Q, g2, degs, b2p)

    return out


# speedup vs baseline: 21.9658x; 2.4637x over previous
import functools
import math

import jax
import jax.numpy as jnp
from jax import lax
from jax.experimental import pallas as pl
from jax.experimental.pallas import tpu as pltpu
from jax.experimental.pallas import tpu_sc as plsc

CH = 128
NC = 2
NS = 16
NW = NC * NS


def _stripe_chunks(stripe):
    out = []
    off = 0
    while off < stripe:
        ln = min(CH, stripe - off)
        out.append((off, ln))
        off += ln
    return out


RING = 2


def _make_sc_agg(n_acc, k_per_tile, width, gather):
    stripe = n_acc // NS
    chunks = _stripe_chunks(stripe)
    mesh = plsc.VectorSubcoreMesh(core_axis_name="c", subcore_axis_name="s")
    nbuf = RING if gather else 1
    idxb = k_per_tile // 2 if gather else k_per_tile
    assert idxb % RING == 0 and k_per_tile % 8 == 0

    scratch = [pltpu.VMEM((idxb, CH), jnp.int32)]
    if gather:
        scratch.append(pltpu.VMEM((idxb, CH), jnp.int32))
    scratch += [pltpu.VMEM((CH, width), jnp.float32)] * nbuf
    scratch.append(pltpu.VMEM_SHARED((n_acc, width), jnp.float32))
    scratch += [pltpu.SemaphoreType.DMA] * (2 * nbuf)

    def body_common(src_hbm, dst_hbm, table_hbm, zeros_hbm, out_hbm,
                    idxd_v, idxs_v, bufs, acc, gsem, ssem):
        c = lax.axis_index("c")
        s = lax.axis_index("s")
        wid = c * NS + s
        base = wid * k_per_tile
        stripe0 = s * stripe

        pltpu.sync_copy(zeros_hbm, bufs[0])
        for off, ln in chunks:
            pltpu.sync_copy(bufs[0].at[pl.ds(0, ln)],
                            acc.at[pl.ds(stripe0 + off, ln)])

        def drain(buf, sem):
            pltpu.make_async_copy(zeros_hbm, buf, sem).wait()

        if gather:
            plsc.subcore_barrier()
            for blk in range(k_per_tile // idxb):
                blk_base = base + blk * idxb
                pltpu.sync_copy(dst_hbm.at[pl.ds(blk_base, idxb)], idxd_v)
                pltpu.sync_copy(src_hbm.at[pl.ds(blk_base, idxb)], idxs_v)

                for b in range(RING):
                    pltpu.async_copy(table_hbm.at[idxs_v.at[b]],
                                     bufs[b], gsem[b])

                def step(g, carry):
                    j0 = g * RING
                    for b in range(RING):
                        drain(bufs[b], gsem[b])
                        pltpu.async_copy(bufs[b], acc.at[idxd_v.at[j0 + b]],
                                         ssem[b], add=True)
                    for b in range(RING):
                        drain(bufs[b], ssem[b])
                        pltpu.async_copy(table_hbm.at[idxs_v.at[j0 + RING + b]],
                                         bufs[b], gsem[b])
                    return carry

                lax.fori_loop(0, idxb // RING - 1, step, 0)
                j0 = idxb - RING
                for b in range(RING):
                    drain(bufs[b], gsem[b])
                    pltpu.async_copy(bufs[b], acc.at[idxd_v.at[j0 + b]],
                                     ssem[b], add=True)
                for b in range(RING):
                    drain(bufs[b], ssem[b])
        else:
            pltpu.sync_copy(dst_hbm.at[pl.ds(base, k_per_tile)], idxd_v)
            pltpu.sync_copy(table_hbm, bufs[0])
            plsc.subcore_barrier()
            fire = 8

            def step(g, carry):
                j0 = g * fire
                for b in range(fire):
                    pltpu.async_copy(bufs[0], acc.at[idxd_v.at[j0 + b]],
                                     ssem[0], add=True)
                for b in range(fire):
                    drain(bufs[0], ssem[0])
                return carry

            lax.fori_loop(0, k_per_tile // fire, step, 0)

        plsc.subcore_barrier()

        out_base = c * n_acc + stripe0
        for off, ln in chunks:
            pltpu.sync_copy(acc.at[pl.ds(stripe0 + off, ln)],
                            bufs[0].at[pl.ds(0, ln)])
            pltpu.sync_copy(bufs[0].at[pl.ds(0, ln)],
                            out_hbm.at[pl.ds(out_base + off, ln)])

    if gather:
        def body(src_hbm, dst_hbm, table_hbm, zeros_hbm, out_hbm,
                 idxd_v, idxs_v, b0, b1, acc, g0, g1, s0, s1):
            body_common(src_hbm, dst_hbm, table_hbm, zeros_hbm, out_hbm,
                        idxd_v, idxs_v, [b0, b1], acc, [g0, g1], [s0, s1])
    else:
        def body(dst_hbm, table_hbm, zeros_hbm, out_hbm,
                 idxd_v, b0, acc, g0, s0):
            body_common(None, dst_hbm, table_hbm, zeros_hbm, out_hbm,
                        idxd_v, None, [b0], acc, [g0], [s0])

    return pl.kernel(
        body,
        out_type=jax.ShapeDtypeStruct((NC * n_acc, width), jnp.float32),
        mesh=mesh,
        scratch_types=scratch,
    )


def _dinv_from(degs):
    deg = degs[0, :, 0:1] + degs[1, :, 0:1] + 1.0
    return lax.rsqrt(deg)


def _tc_b_body(x_ref, w1_ref, degs_ref, g1_ref):
    dinv = _dinv_from(degs_ref[...])
    h = jnp.dot(x_ref[...], w1_ref[...],
                preferred_element_type=jnp.float32,
                precision=lax.Precision.HIGHEST)
    g1_ref[...] = dinv * h


def _tc_d_body(p_ref, g1_ref, degs_ref, w2_ref, b1_ref, g2_ref):
    dinv = _dinv_from(degs_ref[...])
    p = p_ref[...]
    agg = p[0] + p[1] + g1_ref[...]
    out1 = dinv * agg + b1_ref[...]
    h2 = jnp.dot(out1, w2_ref[...],
                 preferred_element_type=jnp.float32,
                 precision=lax.Precision.HIGHEST)
    g2_ref[...] = dinv * h2


def _tc_f_body(q_ref, g2_ref, degs_ref, b2_ref, out_ref):
    dinv = _dinv_from(degs_ref[...])
    q = q_ref[...]
    d_out = out_ref.shape[1]
    z = (dinv * (q[0] + q[1] + g2_ref[...]) + b2_ref[...])[:, :d_out]
    zmax = jnp.max(z, axis=1, keepdims=True)
    zmin = jnp.min(z, axis=1, keepdims=True)
    s = (z - zmin) / (zmax - zmin)
    nrm = jnp.sqrt(jnp.sum(s * s, axis=1, keepdims=True))
    out_ref[...] = s / jnp.maximum(nrm, 1e-12)


@jax.jit
def kernel(x, edge_index, W1, b1, W2, b2):
    n, d_in = x.shape
    d_hid = W1.shape[1]
    d_out = W2.shape[1]
    e = edge_index.shape[1]

    k_per_tile = -(-(-(-e // (CH * NW))) // 8) * 8
    e_pad = k_per_tile * CH * NW
    stripe = -(-(n + 112) // (NS * 8)) * 8
    n_acc = NS * stripe
    pad_rows = n_acc - n

    src = edge_index[0].astype(jnp.int32)
    dst = edge_index[1].astype(jnp.int32)
    npad = e_pad - e
    src_p = jnp.concatenate([src, jnp.arange(npad, dtype=jnp.int32) % n])
    dst_p = jnp.concatenate(
        [dst, n + (jnp.arange(npad, dtype=jnp.int32) % pad_rows)])
    src2 = src_p.reshape(NW * k_per_tile, CH)
    dst2 = dst_p.reshape(NW * k_per_tile, CH)

    dp = 128
    W2p = jnp.zeros((d_hid, dp), jnp.float32).at[:, :d_out].set(W2)
    b2p = jnp.zeros((1, dp), jnp.float32).at[0, :d_out].set(b2)

    ones_blk = jnp.zeros((CH, dp), jnp.float32).at[:, 0].set(1.0)
    zeros_h = jnp.zeros((CH, d_hid), jnp.float32)
    zeros_p = jnp.zeros((CH, dp), jnp.float32)

    deg_k = _make_sc_agg(n_acc, k_per_tile, dp, gather=False)
    degs = deg_k(dst2, ones_blk, zeros_p).reshape(NC, n_acc, dp)

    bk = 1000
    grid = (n // bk,)
    g1 = pl.pallas_call(
        _tc_b_body,
        grid=grid,
        in_specs=[
            pl.BlockSpec((bk, d_in), lambda i: (i, 0)),
            pl.BlockSpec((d_in, d_hid), lambda i: (0, 0)),
            pl.BlockSpec((NC, bk, 128), lambda i: (0, i, 0)),
        ],
        out_specs=pl.BlockSpec((bk, d_hid), lambda i: (i, 0)),
        out_shape=jax.ShapeDtypeStruct((n, d_hid), jnp.float32),
    )(x, W1, degs)

    agg1_k = _make_sc_agg(n_acc, k_per_tile, d_hid, gather=True)
    P = agg1_k(src2, dst2, g1, zeros_h).reshape(NC, n_acc, d_hid)

    g2 = pl.pallas_call(
        _tc_d_body,
        grid=grid,
        in_specs=[
            pl.BlockSpec((NC, bk, d_hid), lambda i: (0, i, 0)),
            pl.BlockSpec((bk, d_hid), lambda i: (i, 0)),
            pl.BlockSpec((NC, bk, 128), lambda i: (0, i, 0)),
            pl.BlockSpec((d_hid, dp), lambda i: (0, 0)),
            pl.BlockSpec((1, d_hid), lambda i: (0, 0)),
        ],
        out_specs=pl.BlockSpec((bk, dp), lambda i: (i, 0)),
        out_shape=jax.ShapeDtypeStruct((n, dp), jnp.float32),
    )(P, g1, degs, W2p, b1.reshape(1, d_hid))

    agg2_k = _make_sc_agg(n_acc, k_per_tile, dp, gather=True)
    Q = agg2_k(src2, dst2, g2, zeros_p).reshape(NC, n_acc, dp)

    out = pl.pallas_call(
        _tc_f_body,
        grid=grid,
        in_specs=[
            pl.BlockSpec((NC, bk, dp), lambda i: (0, i, 0)),
            pl.BlockSpec((bk, dp), lambda i: (i, 0)),
            pl.BlockSpec((NC, bk, 128), lambda i: (0, i, 0)),
            pl.BlockSpec((1, dp), lambda i: (0, 0)),
        ],
        out_specs=pl.BlockSpec((bk, d_out), lambda i: (i, 0)),
        out_shape=jax.ShapeDtypeStruct((n, d_out), jnp.float32),
    )(Q, g2, degs, b2p)

    return out


# speedup vs baseline: 21.9702x; 1.0002x over previous
import functools
import math

import jax
import jax.numpy as jnp
from jax import lax
from jax.experimental import pallas as pl
from jax.experimental.pallas import tpu as pltpu
from jax.experimental.pallas import tpu_sc as plsc

CH = 128
NC = 2
NS = 16
NW = NC * NS


def _stripe_chunks(stripe):
    out = []
    off = 0
    while off < stripe:
        ln = min(CH, stripe - off)
        out.append((off, ln))
        off += ln
    return out


RING = 2


def _make_sc_agg(n_acc, k_per_tile, width, gather):
    stripe = n_acc // NS
    chunks = _stripe_chunks(stripe)
    mesh = plsc.VectorSubcoreMesh(core_axis_name="c", subcore_axis_name="s")
    nbuf = RING if gather else 1
    idxb = k_per_tile // 2 if gather else k_per_tile
    assert idxb % RING == 0 and k_per_tile % 8 == 0

    scratch = [pltpu.VMEM((idxb, CH), jnp.int32)]
    if gather:
        scratch.append(pltpu.VMEM((idxb, CH), jnp.int32))
    scratch += [pltpu.VMEM((CH, width), jnp.float32)] * nbuf
    scratch.append(pltpu.VMEM_SHARED((n_acc, width), jnp.float32))
    scratch += [pltpu.SemaphoreType.DMA] * (2 * nbuf)

    def body_common(src_hbm, dst_hbm, table_hbm, zeros_hbm, out_hbm,
                    idxd_v, idxs_v, bufs, acc, gsem, ssem):
        c = lax.axis_index("c")
        s = lax.axis_index("s")
        wid = c * NS + s
        base = wid * k_per_tile
        stripe0 = s * stripe

        pltpu.sync_copy(zeros_hbm, bufs[0])
        for off, ln in chunks:
            pltpu.sync_copy(bufs[0].at[pl.ds(0, ln)],
                            acc.at[pl.ds(stripe0 + off, ln)])

        def drain(buf, sem):
            pltpu.make_async_copy(zeros_hbm, buf, sem).wait()

        if gather:
            plsc.subcore_barrier()
            for blk in range(k_per_tile // idxb):
                blk_base = base + blk * idxb
                pltpu.sync_copy(dst_hbm.at[pl.ds(blk_base, idxb)], idxd_v)
                pltpu.sync_copy(src_hbm.at[pl.ds(blk_base, idxb)], idxs_v)

                pltpu.sync_copy(zeros_hbm, bufs[1])
                pltpu.async_copy(table_hbm.at[idxs_v.at[0]], bufs[0], gsem[0])
                pltpu.async_copy(bufs[1], acc.at[idxd_v.at[0]],
                                 ssem[1], add=True)

                def half(j0, j1, j2):
                    drain(bufs[1], ssem[1])
                    pltpu.async_copy(table_hbm.at[idxs_v.at[j1]],
                                     bufs[1], gsem[1])
                    drain(bufs[0], gsem[0])
                    pltpu.async_copy(bufs[0], acc.at[idxd_v.at[j0]],
                                     ssem[0], add=True)
                    drain(bufs[1], gsem[1])
                    pltpu.async_copy(bufs[1], acc.at[idxd_v.at[j1]],
                                     ssem[1], add=True)
                    drain(bufs[0], ssem[0])
                    if j2 is not None:
                        pltpu.async_copy(table_hbm.at[idxs_v.at[j2]],
                                         bufs[0], gsem[0])

                def step(g, carry):
                    j = 2 * g
                    half(j, j + 1, j + 2)
                    return carry

                lax.fori_loop(0, idxb // 2 - 1, step, 0)
                half(idxb - 2, idxb - 1, None)
                drain(bufs[1], ssem[1])
        else:
            pltpu.sync_copy(dst_hbm.at[pl.ds(base, k_per_tile)], idxd_v)
            pltpu.sync_copy(table_hbm, bufs[0])
            plsc.subcore_barrier()
            fire = 8

            def step(g, carry):
                j0 = g * fire
                for b in range(fire):
                    pltpu.async_copy(bufs[0], acc.at[idxd_v.at[j0 + b]],
                                     ssem[0], add=True)
                for b in range(fire):
                    drain(bufs[0], ssem[0])
                return carry

            lax.fori_loop(0, k_per_tile // fire, step, 0)

        plsc.subcore_barrier()

        out_base = c * n_acc + stripe0
        for off, ln in chunks:
            pltpu.sync_copy(acc.at[pl.ds(stripe0 + off, ln)],
                            bufs[0].at[pl.ds(0, ln)])
            pltpu.sync_copy(bufs[0].at[pl.ds(0, ln)],
                            out_hbm.at[pl.ds(out_base + off, ln)])

    if gather:
        def body(src_hbm, dst_hbm, table_hbm, zeros_hbm, out_hbm,
                 idxd_v, idxs_v, b0, b1, acc, g0, g1, s0, s1):
            body_common(src_hbm, dst_hbm, table_hbm, zeros_hbm, out_hbm,
                        idxd_v, idxs_v, [b0, b1], acc, [g0, g1], [s0, s1])
    else:
        def body(dst_hbm, table_hbm, zeros_hbm, out_hbm,
                 idxd_v, b0, acc, g0, s0):
            body_common(None, dst_hbm, table_hbm, zeros_hbm, out_hbm,
                        idxd_v, None, [b0], acc, [g0], [s0])

    return pl.kernel(
        body,
        out_type=jax.ShapeDtypeStruct((NC * n_acc, width), jnp.float32),
        mesh=mesh,
        scratch_types=scratch,
    )


def _dinv_from(degs):
    deg = degs[0, :, 0:1] + degs[1, :, 0:1] + 1.0
    return lax.rsqrt(deg)


def _tc_b_body(x_ref, w1_ref, degs_ref, g1_ref):
    dinv = _dinv_from(degs_ref[...])
    h = jnp.dot(x_ref[...], w1_ref[...],
                preferred_element_type=jnp.float32,
                precision=lax.Precision.HIGHEST)
    g1_ref[...] = dinv * h


def _tc_d_body(p_ref, g1_ref, degs_ref, w2_ref, b1_ref, g2_ref):
    dinv = _dinv_from(degs_ref[...])
    p = p_ref[...]
    agg = p[0] + p[1] + g1_ref[...]
    out1 = dinv * agg + b1_ref[...]
    h2 = jnp.dot(out1, w2_ref[...],
                 preferred_element_type=jnp.float32,
                 precision=lax.Precision.HIGHEST)
    g2_ref[...] = dinv * h2


def _tc_f_body(q_ref, g2_ref, degs_ref, b2_ref, out_ref):
    dinv = _dinv_from(degs_ref[...])
    q = q_ref[...]
    d_out = out_ref.shape[1]
    z = (dinv * (q[0] + q[1] + g2_ref[...]) + b2_ref[...])[:, :d_out]
    zmax = jnp.max(z, axis=1, keepdims=True)
    zmin = jnp.min(z, axis=1, keepdims=True)
    s = (z - zmin) / (zmax - zmin)
    nrm = jnp.sqrt(jnp.sum(s * s, axis=1, keepdims=True))
    out_ref[...] = s / jnp.maximum(nrm, 1e-12)


@jax.jit
def kernel(x, edge_index, W1, b1, W2, b2):
    n, d_in = x.shape
    d_hid = W1.shape[1]
    d_out = W2.shape[1]
    e = edge_index.shape[1]

    k_per_tile = -(-(-(-e // (CH * NW))) // 8) * 8
    e_pad = k_per_tile * CH * NW
    stripe = -(-(n + 112) // (NS * 8)) * 8
    n_acc = NS * stripe
    pad_rows = n_acc - n

    src = edge_index[0].astype(jnp.int32)
    dst = edge_index[1].astype(jnp.int32)
    npad = e_pad - e
    src_p = jnp.concatenate([src, jnp.arange(npad, dtype=jnp.int32) % n])
    dst_p = jnp.concatenate(
        [dst, n + (jnp.arange(npad, dtype=jnp.int32) % pad_rows)])
    src2 = src_p.reshape(NW * k_per_tile, CH)
    dst2 = dst_p.reshape(NW * k_per_tile, CH)

    dp = 128
    W2p = jnp.zeros((d_hid, dp), jnp.float32).at[:, :d_out].set(W2)
    b2p = jnp.zeros((1, dp), jnp.float32).at[0, :d_out].set(b2)

    dw = 128
    ones_blk = jnp.zeros((CH, dw), jnp.float32).at[:, 0].set(1.0)
    zeros_h = jnp.zeros((CH, d_hid), jnp.float32)
    zeros_p = jnp.zeros((CH, dp), jnp.float32)

    deg_k = _make_sc_agg(n_acc, k_per_tile, dw, gather=False)
    degs = deg_k(dst2, ones_blk, zeros_p).reshape(NC, n_acc, dw)

    bk = 1000
    grid = (n // bk,)
    g1 = pl.pallas_call(
        _tc_b_body,
        grid=grid,
        in_specs=[
            pl.BlockSpec((bk, d_in), lambda i: (i, 0)),
            pl.BlockSpec((d_in, d_hid), lambda i: (0, 0)),
            pl.BlockSpec((NC, bk, 128), lambda i: (0, i, 0)),
        ],
        out_specs=pl.BlockSpec((bk, d_hid), lambda i: (i, 0)),
        out_shape=jax.ShapeDtypeStruct((n, d_hid), jnp.float32),
    )(x, W1, degs)

    agg1_k = _make_sc_agg(n_acc, k_per_tile, d_hid, gather=True)
    P = agg1_k(src2, dst2, g1, zeros_h).reshape(NC, n_acc, d_hid)

    g2 = pl.pallas_call(
        _tc_d_body,
        grid=grid,
        in_specs=[
            pl.BlockSpec((NC, bk, d_hid), lambda i: (0, i, 0)),
            pl.BlockSpec((bk, d_hid), lambda i: (i, 0)),
            pl.BlockSpec((NC, bk, 128), lambda i: (0, i, 0)),
            pl.BlockSpec((d_hid, dp), lambda i: (0, 0)),
            pl.BlockSpec((1, d_hid), lambda i: (0, 0)),
        ],
        out_specs=pl.BlockSpec((bk, dp), lambda i: (i, 0)),
        out_shape=jax.ShapeDtypeStruct((n, dp), jnp.float32),
    )(P, g1, degs, W2p, b1.reshape(1, d_hid))

    agg2_k = _make_sc_agg(n_acc, k_per_tile, dp, gather=True)
    Q = agg2_k(src2, dst2, g2, zeros_p).reshape(NC, n_acc, dp)

    out = pl.pallas_call(
        _tc_f_body,
        grid=grid,
        in_specs=[
            pl.BlockSpec((NC, bk, dp), lambda i: (0, i, 0)),
            pl.BlockSpec((bk, dp), lambda i: (i, 0)),
            pl.BlockSpec((NC, bk, 128), lambda i: (0, i, 0)),
            pl.BlockSpec((1, dp), lambda i: (0, 0)),
        ],
        out_specs=pl.BlockSpec((bk, d_out), lambda i: (i, 0)),
        out_shape=jax.ShapeDtypeStruct((n, d_out), jnp.float32),
    )(Q, g2, degs, b2p)

    return out


# speedup vs baseline: 22.0033x; 1.0015x over previous
import functools
import math

import jax
import jax.numpy as jnp
from jax import lax
from jax.experimental import pallas as pl
from jax.experimental.pallas import tpu as pltpu
from jax.experimental.pallas import tpu_sc as plsc

CH = 128
NC = 2
NS = 16
NW = NC * NS


def _stripe_chunks(stripe):
    out = []
    off = 0
    while off < stripe:
        ln = min(CH, stripe - off)
        out.append((off, ln))
        off += ln
    return out


RING = 2


def _make_sc_agg(n_acc, k_per_tile, width, gather, dtype=jnp.float32):
    stripe = n_acc // NS
    chunks = _stripe_chunks(stripe)
    mesh = plsc.VectorSubcoreMesh(core_axis_name="c", subcore_axis_name="s")
    nbuf = RING if gather else 1
    idxb = k_per_tile // 2 if gather else k_per_tile
    assert idxb % RING == 0 and k_per_tile % 8 == 0

    scratch = [pltpu.VMEM((idxb, CH), jnp.int32)]
    if gather:
        scratch.append(pltpu.VMEM((idxb, CH), jnp.int32))
    scratch += [pltpu.VMEM((CH, width), dtype)] * nbuf
    scratch.append(pltpu.VMEM_SHARED((n_acc, width), dtype))
    scratch += [pltpu.SemaphoreType.DMA] * (2 * nbuf)

    def body_common(src_hbm, dst_hbm, table_hbm, zeros_hbm, out_hbm,
                    idxd_v, idxs_v, bufs, acc, gsem, ssem):
        c = lax.axis_index("c")
        s = lax.axis_index("s")
        wid = c * NS + s
        base = wid * k_per_tile
        stripe0 = s * stripe

        pltpu.sync_copy(zeros_hbm, bufs[0])
        for off, ln in chunks:
            pltpu.sync_copy(bufs[0].at[pl.ds(0, ln)],
                            acc.at[pl.ds(stripe0 + off, ln)])

        def drain(buf, sem):
            pltpu.make_async_copy(zeros_hbm, buf, sem).wait()

        if gather:
            plsc.subcore_barrier()
            for blk in range(k_per_tile // idxb):
                blk_base = base + blk * idxb
                pltpu.sync_copy(dst_hbm.at[pl.ds(blk_base, idxb)], idxd_v)
                pltpu.sync_copy(src_hbm.at[pl.ds(blk_base, idxb)], idxs_v)

                pltpu.sync_copy(zeros_hbm, bufs[1])
                pltpu.async_copy(table_hbm.at[idxs_v.at[0]], bufs[0], gsem[0])
                pltpu.async_copy(bufs[1], acc.at[idxd_v.at[0]],
                                 ssem[1], add=True)

                def half(j0, j1, j2):
                    drain(bufs[1], ssem[1])
                    pltpu.async_copy(table_hbm.at[idxs_v.at[j1]],
                                     bufs[1], gsem[1])
                    drain(bufs[0], gsem[0])
                    pltpu.async_copy(bufs[0], acc.at[idxd_v.at[j0]],
                                     ssem[0], add=True)
                    drain(bufs[1], gsem[1])
                    pltpu.async_copy(bufs[1], acc.at[idxd_v.at[j1]],
                                     ssem[1], add=True)
                    drain(bufs[0], ssem[0])
                    if j2 is not None:
                        pltpu.async_copy(table_hbm.at[idxs_v.at[j2]],
                                         bufs[0], gsem[0])

                def step(g, carry):
                    j = 2 * g
                    half(j, j + 1, j + 2)
                    return carry

                lax.fori_loop(0, idxb // 2 - 1, step, 0)
                half(idxb - 2, idxb - 1, None)
                drain(bufs[1], ssem[1])
        else:
            pltpu.sync_copy(dst_hbm.at[pl.ds(base, k_per_tile)], idxd_v)
            pltpu.sync_copy(table_hbm, bufs[0])
            plsc.subcore_barrier()
            fire = 8

            def step(g, carry):
                j0 = g * fire
                for b in range(fire):
                    pltpu.async_copy(bufs[0], acc.at[idxd_v.at[j0 + b]],
                                     ssem[0], add=True)
                for b in range(fire):
                    drain(bufs[0], ssem[0])
                return carry

            lax.fori_loop(0, k_per_tile // fire, step, 0)

        plsc.subcore_barrier()

        out_base = c * n_acc + stripe0
        for off, ln in chunks:
            pltpu.sync_copy(acc.at[pl.ds(stripe0 + off, ln)],
                            bufs[0].at[pl.ds(0, ln)])
            pltpu.sync_copy(bufs[0].at[pl.ds(0, ln)],
                            out_hbm.at[pl.ds(out_base + off, ln)])

    if gather:
        def body(src_hbm, dst_hbm, table_hbm, zeros_hbm, out_hbm,
                 idxd_v, idxs_v, b0, b1, acc, g0, g1, s0, s1):
            body_common(src_hbm, dst_hbm, table_hbm, zeros_hbm, out_hbm,
                        idxd_v, idxs_v, [b0, b1], acc, [g0, g1], [s0, s1])
    else:
        def body(dst_hbm, table_hbm, zeros_hbm, out_hbm,
                 idxd_v, b0, acc, g0, s0):
            body_common(None, dst_hbm, table_hbm, zeros_hbm, out_hbm,
                        idxd_v, None, [b0], acc, [g0], [s0])

    return pl.kernel(
        body,
        out_type=jax.ShapeDtypeStruct((NC * n_acc, width), dtype),
        mesh=mesh,
        scratch_types=scratch,
    )


def _dinv_from(degs):
    deg = degs[0, :, 0:1] + degs[1, :, 0:1] + 1.0
    return lax.rsqrt(deg)


def _tc_b_body(x_ref, w1_ref, degs_ref, g1_ref):
    dinv = _dinv_from(degs_ref[...])
    h = jnp.dot(x_ref[...], w1_ref[...],
                preferred_element_type=jnp.float32,
                precision=lax.Precision.HIGHEST)
    g1_ref[...] = dinv * h


def _tc_d_body(p_ref, g1_ref, degs_ref, w2_ref, b1_ref, g2_ref):
    dinv = _dinv_from(degs_ref[...])
    p = p_ref[...]
    agg = p[0] + p[1] + g1_ref[...]
    out1 = dinv * agg + b1_ref[...]
    h2 = jnp.dot(out1, w2_ref[...],
                 preferred_element_type=jnp.float32,
                 precision=lax.Precision.HIGHEST)
    g2_ref[...] = (dinv * h2).astype(g2_ref.dtype)


def _tc_f_body(q_ref, g2_ref, degs_ref, b2_ref, out_ref):
    dinv = _dinv_from(degs_ref[...])
    q = q_ref[...].astype(jnp.float32)
    g2f = g2_ref[...].astype(jnp.float32)
    d_out = out_ref.shape[1]
    z = (dinv * (q[0] + q[1] + g2f) + b2_ref[...])[:, :d_out]
    zmax = jnp.max(z, axis=1, keepdims=True)
    zmin = jnp.min(z, axis=1, keepdims=True)
    s = (z - zmin) / (zmax - zmin)
    nrm = jnp.sqrt(jnp.sum(s * s, axis=1, keepdims=True))
    out_ref[...] = s / jnp.maximum(nrm, 1e-12)


@jax.jit
def kernel(x, edge_index, W1, b1, W2, b2):
    n, d_in = x.shape
    d_hid = W1.shape[1]
    d_out = W2.shape[1]
    e = edge_index.shape[1]

    k_per_tile = -(-(-(-e // (CH * NW))) // 8) * 8
    e_pad = k_per_tile * CH * NW
    stripe = -(-(n + 112) // (NS * 8)) * 8
    n_acc = NS * stripe
    pad_rows = n_acc - n

    src = edge_index[0].astype(jnp.int32)
    dst = edge_index[1].astype(jnp.int32)
    npad = e_pad - e
    src_p = jnp.concatenate([src, jnp.arange(npad, dtype=jnp.int32) % n])
    dst_p = jnp.concatenate(
        [dst, n + (jnp.arange(npad, dtype=jnp.int32) % pad_rows)])
    src2 = src_p.reshape(NW * k_per_tile, CH)
    dst2 = dst_p.reshape(NW * k_per_tile, CH)

    dp = 128
    W2p = jnp.zeros((d_hid, dp), jnp.float32).at[:, :d_out].set(W2)
    b2p = jnp.zeros((1, dp), jnp.float32).at[0, :d_out].set(b2)

    dw = 128
    ones_blk = jnp.zeros((CH, dw), jnp.float32).at[:, 0].set(1.0)
    zeros_h = jnp.zeros((CH, d_hid), jnp.float32)
    zeros_p = jnp.zeros((CH, dp), jnp.float32)

    deg_k = _make_sc_agg(n_acc, k_per_tile, dw, gather=False)
    degs = deg_k(dst2, ones_blk, zeros_p).reshape(NC, n_acc, dw)

    bk = 1000
    grid = (n // bk,)
    g1 = pl.pallas_call(
        _tc_b_body,
        grid=grid,
        in_specs=[
            pl.BlockSpec((bk, d_in), lambda i: (i, 0)),
            pl.BlockSpec((d_in, d_hid), lambda i: (0, 0)),
            pl.BlockSpec((NC, bk, 128), lambda i: (0, i, 0)),
        ],
        out_specs=pl.BlockSpec((bk, d_hid), lambda i: (i, 0)),
        out_shape=jax.ShapeDtypeStruct((n, d_hid), jnp.float32),
    )(x, W1, degs)

    agg1_k = _make_sc_agg(n_acc, k_per_tile, d_hid, gather=True)
    P = agg1_k(src2, dst2, g1, zeros_h).reshape(NC, n_acc, d_hid)

    g2 = pl.pallas_call(
        _tc_d_body,
        grid=grid,
        in_specs=[
            pl.BlockSpec((NC, bk, d_hid), lambda i: (0, i, 0)),
            pl.BlockSpec((bk, d_hid), lambda i: (i, 0)),
            pl.BlockSpec((NC, bk, 128), lambda i: (0, i, 0)),
            pl.BlockSpec((d_hid, dp), lambda i: (0, 0)),
            pl.BlockSpec((1, d_hid), lambda i: (0, 0)),
        ],
        out_specs=pl.BlockSpec((bk, dp), lambda i: (i, 0)),
        out_shape=jax.ShapeDtypeStruct((n, dp), jnp.float32),
    )(P, g1, degs, W2p, b1.reshape(1, d_hid))

    agg2_k = _make_sc_agg(n_acc, k_per_tile, dp, gather=True)
    Q = agg2_k(src2, dst2, g2, zeros_p).reshape(NC, n_acc, dp)

    out = pl.pallas_call(
        _tc_f_body,
        grid=grid,
        in_specs=[
            pl.BlockSpec((NC, bk, dp), lambda i: (0, i, 0)),
            pl.BlockSpec((bk, dp), lambda i: (i, 0)),
            pl.BlockSpec((NC, bk, 128), lambda i: (0, i, 0)),
            pl.BlockSpec((1, dp), lambda i: (0, 0)),
        ],
        out_specs=pl.BlockSpec((bk, d_out), lambda i: (i, 0)),
        out_shape=jax.ShapeDtypeStruct((n, d_out), jnp.float32),
    )(Q, g2, degs, b2p)

    return out
